# Initial kernel scaffold; baseline (speedup 1.0000x reference)
import jax
import jax.numpy as jnp
from jax.experimental import pallas as pl


def kernel(entity_ids, entity_table, pos_table, entity_id_to_pos_index, W, b):
    raise NotImplementedError("write your pallas kernel here")


# speedup vs baseline: 6.5112x; 6.5112x over previous
import functools

import jax
import jax.numpy as jnp
from jax import lax
from jax.experimental import pallas as pl
from jax.experimental.pallas import tpu as pltpu
from jax.experimental.pallas import tpu_sc as plsc

N = 61440
D = 512
NC, NS = 2, 16
NW = NC * NS
B_PER_W = N // NW
CHUNK = 128
N_CHUNKS = B_PER_W // CHUNK

BLK = 1024
N_BLKS = N // BLK


def _sc_gather(ids, table, posmap):
    mesh = plsc.VectorSubcoreMesh(core_axis_name="c", subcore_axis_name="s")

    @functools.partial(
        pl.kernel,
        mesh=mesh,
        out_type=(
            jax.ShapeDtypeStruct((N, D), jnp.float32),
            jax.ShapeDtypeStruct((N,), jnp.int32),
        ),
        scratch_types=[
            pltpu.VMEM((CHUNK,), jnp.int32),
            pltpu.VMEM((CHUNK, D), jnp.float32),
            pltpu.VMEM((CHUNK,), jnp.int32),
            pltpu.SemaphoreType.DMA,
            pltpu.SemaphoreType.DMA,
        ],
    )
    def k(ids_hbm, table_hbm, posmap_hbm, rows_out, pos_out,
          idx_v, rows_v, pos_v, sem_r, sem_p):
        wid = lax.axis_index("s") * NC + lax.axis_index("c")
        base = wid * B_PER_W

        def body(ch, carry):
            off = base + ch * CHUNK
            pltpu.sync_copy(ids_hbm.at[pl.ds(off, CHUNK)], idx_v)
            cp_r = pltpu.async_copy(table_hbm.at[idx_v], rows_v, sem_r)
            cp_p = pltpu.async_copy(posmap_hbm.at[idx_v], pos_v, sem_p)
            cp_r.wait()
            cp_p.wait()
            pltpu.sync_copy(rows_v, rows_out.at[pl.ds(off, CHUNK)])
            pltpu.sync_copy(pos_v, pos_out.at[pl.ds(off, CHUNK)])
            return carry

        lax.fori_loop(0, N_CHUNKS, body, 0)

    return k(ids, table, posmap)


def _tc_body(g_ref, pos_ref, we_ref, pos16_ref, wp_ref, b_ref, out_ref):
    p16 = lax.dot_general(
        pos16_ref[...], wp_ref[...], (((1,), (1,)), ((), ())),
        preferred_element_type=jnp.float32) + b_ref[...]
    pos = pos_ref[0, 0, :]
    onehot = (pos[:, None] == lax.broadcasted_iota(
        jnp.int32, (BLK, 16), 1)).astype(jnp.float32)
    out_ref[...] = (
        lax.dot_general(g_ref[...], we_ref[...], (((1,), (1,)), ((), ())),
                        preferred_element_type=jnp.float32)
        + jnp.dot(onehot, p16, preferred_element_type=jnp.float32))


def kernel(entity_ids, entity_table, pos_table, entity_id_to_pos_index, W, b):
    ids = entity_ids.reshape(-1).astype(jnp.int32)
    posmap = entity_id_to_pos_index.astype(jnp.int32)

    rows, pos = _sc_gather(ids, entity_table, posmap)

    we = W[:, :D]
    wp = jnp.pad(W[:, D:], ((0, 0), (0, 7)))
    pos16 = jnp.pad(pos_table[:16], ((0, 0), (0, 7)))
    b2 = b.reshape(1, D)
    pos3 = pos.reshape(N_BLKS, 1, BLK)

    out = pl.pallas_call(
        _tc_body,
        grid=(N_BLKS,),
        in_specs=[
            pl.BlockSpec((BLK, D), lambda i: (i, 0)),
            pl.BlockSpec((1, 1, BLK), lambda i: (i, 0, 0)),
            pl.BlockSpec((D, D), lambda i: (0, 0)),
            pl.BlockSpec((16, 32), lambda i: (0, 0)),
            pl.BlockSpec((D, 32), lambda i: (0, 0)),
            pl.BlockSpec((1, D), lambda i: (0, 0)),
        ],
        out_specs=pl.BlockSpec((BLK, D), lambda i: (i, 0)),
        out_shape=jax.ShapeDtypeStruct((N, D), jnp.float32),
    )(rows, pos3, we, pos16, wp, b2)

    return out.reshape(*entity_ids.shape, D)
